# Initial kernel scaffold; baseline (speedup 1.0000x reference)
import jax
import jax.numpy as jnp
from jax.experimental import pallas as pl


def kernel(title_ids, text_token_ids, title_table, text_table):
    raise NotImplementedError("write your pallas kernel here")


# speedup vs baseline: 12.9118x; 12.9118x over previous
import functools

import jax
import jax.numpy as jnp
from jax import lax
from jax.experimental import pallas as pl
from jax.experimental.pallas import tpu as pltpu
from jax.experimental.pallas import tpu_sc as plsc

BATCH = 16384
EMBED = 32
SEQ = 20
L = 16

NC = 2
NS = 16
NW = NC * NS
BPW = BATCH // NW
CHUNK = 128
NCHUNK = BPW // CHUNK
TOK_PER_CHUNK = CHUNK * SEQ
IDX_DMA = 128

_mesh = plsc.VectorSubcoreMesh(core_axis_name="c", subcore_axis_name="s")


@functools.partial(
    pl.kernel,
    out_type=jax.ShapeDtypeStruct((BATCH, 2 * EMBED), jnp.float32),
    mesh=_mesh,
    compiler_params=pltpu.CompilerParams(needs_layout_passes=False,
                                          use_tc_tiling_on_sc=False),
    scratch_types=[
        pltpu.VMEM((CHUNK,), jnp.int32),
        pltpu.VMEM((CHUNK, EMBED), jnp.float32),
        pltpu.VMEM((TOK_PER_CHUNK,), jnp.int32),
        pltpu.VMEM((TOK_PER_CHUNK, EMBED), jnp.float32),
        pltpu.VMEM((CHUNK,), jnp.float32),
        pltpu.VMEM((CHUNK,), jnp.float32),
        pltpu.VMEM((1, EMBED), jnp.float32),
        pltpu.VMEM((CHUNK, 2 * EMBED), jnp.float32),
        pltpu.SemaphoreType.DMA,
    ],
)
def _sc_kernel(title_hbm, tok_hbm, ttab_hbm, xtab_hbm, out_hbm,
               tidx_v, trows_v, tokidx_v, tokbuf_v, n0_v, scale_v,
               row0_v, rowbuf_v, sem):
    wid = lax.axis_index("s") * NC + lax.axis_index("c")
    base = wid * BPW

    pltpu.sync_copy(xtab_hbm.at[pl.ds(0, 1)], row0_v)
    r00 = row0_v[0, pl.ds(0, L)]
    r01 = row0_v[0, pl.ds(L, L)]

    for c in range(NCHUNK):
        cbase = base + c * CHUNK

        pltpu.sync_copy(title_hbm.at[pl.ds(cbase, CHUNK)], tidx_v)
        pltpu.sync_copy(tok_hbm.at[pl.ds(cbase * SEQ, TOK_PER_CHUNK)], tokidx_v)

        copies = [pltpu.async_copy(ttab_hbm.at[tidx_v], trows_v, sem)]
        for j in range(TOK_PER_CHUNK // IDX_DMA):
            copies.append(pltpu.async_copy(
                xtab_hbm.at[tokidx_v.at[pl.ds(j * IDX_DMA, IDX_DMA)]],
                tokbuf_v.at[pl.ds(j * IDX_DMA, IDX_DMA)], sem))

        for g in range(CHUNK // L):
            sidx = lax.iota(jnp.int32, L) + (g * L)
            pbase = sidx * SEQ
            zc = jnp.zeros((L,), jnp.float32)
            one = jnp.ones((L,), jnp.float32)
            zero = jnp.zeros((L,), jnp.float32)
            for t in range(SEQ):
                ids = plsc.load_gather(tokidx_v, [pbase + t])
                zc = zc + jnp.where(ids == 0, one, zero)
            n0_v[pl.ds(g * L, L)] = zc
            cnt = jnp.maximum(jnp.full((L,), float(SEQ), jnp.float32) - zc,
                              one)
            scale_v[pl.ds(g * L, L)] = one / cnt

        for cp in copies:
            cp.wait()

        def sample_body(s, _):
            splat = jnp.full((L,), 0, jnp.int32) + s
            sc = plsc.load_gather(scale_v, [splat])
            n0 = plsc.load_gather(n0_v, [splat])
            acc0 = -n0 * r00
            acc1 = -n0 * r01
            rbase = s * SEQ
            for t in range(SEQ):
                acc0 = acc0 + tokbuf_v[rbase + t, pl.ds(0, L)]
                acc1 = acc1 + tokbuf_v[rbase + t, pl.ds(L, L)]
            rowbuf_v[s, pl.ds(0, L)] = trows_v[s, pl.ds(0, L)]
            rowbuf_v[s, pl.ds(L, L)] = trows_v[s, pl.ds(L, L)]
            rowbuf_v[s, pl.ds(2 * L, L)] = acc0 * sc
            rowbuf_v[s, pl.ds(3 * L, L)] = acc1 * sc
            return 0

        lax.fori_loop(0, CHUNK, sample_body, 0)

        pltpu.sync_copy(rowbuf_v, out_hbm.at[pl.ds(cbase, CHUNK)])


def kernel(title_ids, text_token_ids, title_table, text_table):
    tok_flat = text_token_ids.reshape(-1).astype(jnp.int32)
    return _sc_kernel(title_ids.astype(jnp.int32), tok_flat,
                      title_table, text_table)
